# baseline (speedup 1.0000x reference)
import functools
import math

import jax
import jax.numpy as jnp
from jax.experimental import pallas as pl
from jax.experimental.pallas import tpu as pltpu

SMOOTHING = 0.1
CONFIDENCE = 1.0 - SMOOTHING
_BLK = 2000


def _body(a_ref, b_ref, t_ref, o_ref, *, const_term, eps, blk, half):
    i = pl.program_id(0)

    @pl.when(i == 0)
    def _():
        o_ref[0, 0] = jnp.float32(const_term)

    ln2 = math.log(2.0)
    wa = jnp.float32(-eps * ln2)
    wc = jnp.float32(-CONFIDENCE * ln2)
    rowa = i * blk + jax.lax.broadcasted_iota(jnp.int32, a_ref.shape, 0)
    sa = jnp.sum(jnp.where(rowa == t_ref[...], wc, wa) * jnp.log2(a_ref[...]))
    rowb = rowa + half
    sb = jnp.sum(jnp.where(rowb == t_ref[...], wc, wa) * jnp.log2(b_ref[...]))
    o_ref[0, 0] += sa + sb


def kernel(x, target):
    n, size = x.shape
    eps = SMOOTHING / (size - 1)
    const_term = n * ((size - 1) * eps * math.log(eps)
                      + CONFIDENCE * math.log(CONFIDENCE))
    half = size // 2
    nsteps = half // _BLK

    xt = x.T
    body = functools.partial(_body, const_term=const_term, eps=eps,
                             blk=_BLK, half=half)
    out = pl.pallas_call(
        body,
        grid=(nsteps,),
        in_specs=[
            pl.BlockSpec((_BLK, n), lambda i: (i, 0)),
            pl.BlockSpec((_BLK, n), lambda i, _h=nsteps: (i + _h, 0)),
            pl.BlockSpec((1, n), lambda i: (0, 0)),
        ],
        out_specs=pl.BlockSpec(memory_space=pltpu.SMEM),
        out_shape=jax.ShapeDtypeStruct((1, 1), jnp.float32),
        compiler_params=pltpu.CompilerParams(
            dimension_semantics=("arbitrary",),
        ),
    )(xt, xt, target.reshape(1, n))
    return out[0, 0]
